# baseline (speedup 1.0000x reference)
import functools
import math

import jax
import jax.numpy as jnp
from jax import lax
from jax.experimental import pallas as pl
from jax.experimental.pallas import tpu as pltpu
from jax.experimental.pallas import tpu_sc as plsc

EMBED_DIM = 64
SCALE = math.sqrt(EMBED_DIM)

B, T = 4096, 200
N = B * T
NUM_CORES = 2
NUM_SUBCORES = 16
NW = NUM_CORES * NUM_SUBCORES
ROWS_PER_W = N // NW
CHUNK = 128
NCHUNK = ROWS_PER_W // CHUNK
NBUF = 4
NGROUP = NCHUNK // NBUF


def _sc_embedding_lookup(x_flat, table2):
    mesh = plsc.VectorSubcoreMesh(core_axis_name="c", subcore_axis_name="s")

    scratch = (
        [pltpu.VMEM((CHUNK,), jnp.int32)] * NBUF
        + [pltpu.VMEM((CHUNK,), jnp.int32)] * NBUF
        + [pltpu.VMEM((CHUNK, 128), jnp.float32)] * NBUF
        + [pltpu.VMEM((CHUNK // 2, 128), jnp.float32)] * NBUF
        + [pltpu.SemaphoreType.DMA] * NBUF
    )

    @functools.partial(
        pl.kernel,
        mesh=mesh,
        out_type=jax.ShapeDtypeStruct((N // 2, 128), jnp.float32),
        scratch_types=scratch,
    )
    def k(idx_hbm, table_hbm, out_hbm, *sc):
        idxr = sc[0 * NBUF:1 * NBUF]
        idx2 = sc[1 * NBUF:2 * NBUF]
        rows = sc[2 * NBUF:3 * NBUF]
        stage = sc[3 * NBUF:4 * NBUF]
        sems = sc[4 * NBUF:5 * NBUF]
        wid = lax.axis_index("s") * NUM_CORES + lax.axis_index("c")
        base = wid * ROWS_PER_W

        def prep(ci, b):
            off = pl.multiple_of(base + ci * CHUNK, CHUNK)
            pltpu.sync_copy(idx_hbm.at[pl.ds(off, CHUNK)], idxr[b])
            for v in range(CHUNK // 16):
                sl = pl.ds(v * 16, 16)
                idx2[b][sl] = lax.shift_right_logical(idxr[b][sl], 1)

        def fire(b):
            pltpu.async_copy(table_hbm.at[idx2[b]], rows[b], sems[b])

        def drain(b):
            pltpu.make_async_copy(table_hbm.at[idx2[b]], rows[b], sems[b]).wait()

        def extract(b):
            @plsc.parallel_loop(0, CHUNK // 16, step=1)
            def _(p):
                p16 = pl.multiple_of(p * 16, 16)
                p8 = pl.multiple_of(p * 8, 8)
                offs = (idxr[b][pl.ds(p16, 16)] & 1) * 64
                for k in range(16):
                    r = p16 + k
                    q = p8 + k // 2
                    h = (k % 2) * 64
                    off = offs[k]
                    for j in range(EMBED_DIM // 16):
                        stage[b][q, pl.ds(h + j * 16, 16)] = (
                            rows[b][r, pl.ds(off + j * 16, 16)] * SCALE
                        )

        def store(ci, b):
            off = pl.multiple_of((base + ci * CHUNK) // 2, CHUNK // 2)
            dst = out_hbm.at[pl.ds(off, CHUNK // 2)]
            pltpu.sync_copy(stage[b], dst)

        for b in range(NBUF):
            prep(b, b)
            fire(b)

        def group_body(g, _):
            for b in range(NBUF):
                ci = g * NBUF + b
                drain(b)
                extract(b)
                store(ci, b)
                prep(ci + NBUF, b)
                fire(b)
            return 0

        lax.fori_loop(0, NGROUP - 1, group_body, 0)

        for b in range(NBUF):
            ci = (NGROUP - 1) * NBUF + b
            drain(b)
            extract(b)
            store(ci, b)

    return k(x_flat, table2)


def kernel(x, table):
    x_flat = x.reshape(N)
    table2 = table.reshape(table.shape[0] // 2, 128)
    out2 = _sc_embedding_lookup(x_flat, table2)
    return out2.reshape(B, T, EMBED_DIM)
